# speedup vs baseline: 9.2487x; 9.2487x over previous
import functools

import jax
import jax.numpy as jnp
from jax import lax
from jax.experimental import pallas as pl
from jax.experimental.pallas import tpu as pltpu
from jax.experimental.pallas import tpu_sc as plsc

B = 4096
L = 200
EMB = 128

NW = 32
N = B * L
PER_W = N // NW
CHUNK = 128
NCHUNK = PER_W // CHUNK

_mesh = plsc.VectorSubcoreMesh(core_axis_name="c", subcore_axis_name="s")


@functools.partial(
    pl.kernel,
    out_type=jax.ShapeDtypeStruct((N, EMB), jnp.float32),
    mesh=_mesh,
    scratch_types=[
        pltpu.VMEM((NCHUNK, CHUNK), jnp.int32),
        pltpu.VMEM((CHUNK, EMB), jnp.float32),
        pltpu.VMEM((CHUNK, EMB), jnp.float32),
        pltpu.SemaphoreType.DMA,
        pltpu.SemaphoreType.DMA,
    ],
)
def _gather_kernel(idx_hbm, table_hbm, out_hbm, idx_v, rows0, rows1, sem0, sem1):
    wid = lax.axis_index("s") * 2 + lax.axis_index("c")
    base = wid * PER_W

    pltpu.sync_copy(idx_hbm.at[pl.ds(wid * NCHUNK, NCHUNK)], idx_v)

    def issue(g, rows, sem):
        return pltpu.async_copy(table_hbm.at[idx_v.at[g]], rows, sem)

    def wait(rows, sem):
        pltpu.make_async_copy(table_hbm.at[idx_v.at[0]], rows, sem).wait()

    def write(g, rows):
        pltpu.sync_copy(rows, out_hbm.at[pl.ds(base + g * CHUNK, CHUNK)])

    issue(0, rows0, sem0)

    @pl.loop(0, NCHUNK, step=2)
    def _body(g):
        d1 = issue(g + 1, rows1, sem1)
        wait(rows0, sem0)
        write(g, rows0)

        @pl.when(g + 2 < NCHUNK)
        def _():
            issue(g + 2, rows0, sem0)

        d1.wait()
        write(g + 1, rows1)


def kernel(features, table):
    idx = features.reshape(NW * NCHUNK, CHUNK)
    out = _gather_kernel(idx, table)
    return out.reshape(B, L, EMB)


# speedup vs baseline: 9.2597x; 1.0012x over previous
import functools

import jax
import jax.numpy as jnp
from jax import lax
from jax.experimental import pallas as pl
from jax.experimental.pallas import tpu as pltpu
from jax.experimental.pallas import tpu_sc as plsc

B = 4096
L = 200
EMB = 128

NW = 32
N = B * L
PER_W = N // NW
CHUNK = 128
NCHUNK = PER_W // CHUNK
GPB = 2
ROWS = CHUNK * GPB
NPAIR = NCHUNK // GPB

_mesh = plsc.VectorSubcoreMesh(core_axis_name="c", subcore_axis_name="s")


@functools.partial(
    pl.kernel,
    out_type=jax.ShapeDtypeStruct((N, EMB), jnp.float32),
    mesh=_mesh,
    scratch_types=[
        pltpu.VMEM((NCHUNK, CHUNK), jnp.int32),
        pltpu.VMEM((ROWS, EMB), jnp.float32),
        pltpu.VMEM((ROWS, EMB), jnp.float32),
        pltpu.SemaphoreType.DMA,
        pltpu.SemaphoreType.DMA,
    ],
)
def _gather_kernel(idx_hbm, table_hbm, out_hbm, idx_v, rows0, rows1, sem0, sem1):
    wid = lax.axis_index("s") * 2 + lax.axis_index("c")
    base = wid * PER_W

    pltpu.sync_copy(idx_hbm.at[pl.ds(wid * NCHUNK, NCHUNK)], idx_v)

    def issue(p, rows, sem):
        for j in range(GPB):
            pltpu.async_copy(
                table_hbm.at[idx_v.at[p * GPB + j]],
                rows.at[pl.ds(j * CHUNK, CHUNK)],
                sem,
            )

    def wait(rows, sem):
        pltpu.make_async_copy(table_hbm.at[pl.ds(0, ROWS)], rows, sem).wait()

    def write(p, rows):
        pltpu.sync_copy(rows, out_hbm.at[pl.ds(base + p * ROWS, ROWS)])

    issue(0, rows0, sem0)

    @pl.loop(0, NPAIR, step=2)
    def _body(p):
        issue(p + 1, rows1, sem1)
        wait(rows0, sem0)
        write(p, rows0)

        @pl.when(p + 2 < NPAIR)
        def _():
            issue(p + 2, rows0, sem0)

        wait(rows1, sem1)
        write(p + 1, rows1)


def kernel(features, table):
    idx = features.reshape(NW * NCHUNK, CHUNK)
    out = _gather_kernel(idx, table)
    return out.reshape(B, L, EMB)
